# baseline (device time: 315445 ns/iter reference)
import functools

import jax
import jax.numpy as jnp
from jax import lax
from jax.experimental import pallas as pl
from jax.experimental.pallas import tpu as pltpu

NZ = 4
TILE = 512


def kernel(partial, gamma):
    _, m, d = partial.shape
    ch = m // NZ
    hh = ch // 2

    def body(x_ref, g_ref, out_ref, comm_ref, send_sems, recv_sems, copy_sems,
             credit_sem):
        my_x = lax.axis_index("x")
        my_y = lax.axis_index("y")
        my_z = lax.axis_index("z")
        right = (my_z + 1) % NZ
        left = (my_z - 1) % NZ

        barrier = pltpu.get_barrier_semaphore()
        for nbr in (left, right):
            pl.semaphore_signal(
                barrier, inc=1,
                device_id=(my_x, my_y, nbr),
                device_id_type=pl.DeviceIdType.MESH,
            )
        pl.semaphore_wait(barrier, 2)

        def load_halves(c_top, c_bot):
            top = pltpu.make_async_copy(
                x_ref.at[0, pl.ds(c_top * ch, hh), :],
                out_ref.at[pl.ds(0, hh), :],
                copy_sems.at[0],
            )
            bot = pltpu.make_async_copy(
                x_ref.at[0, pl.ds(c_bot * ch + hh, hh), :],
                out_ref.at[pl.ds(hh, hh), :],
                copy_sems.at[1],
            )
            top.start()
            bot.start()
            top.wait()
            bot.wait()

        load_halves((my_z - 1) % NZ, (my_z + 1) % NZ)
        for t in range(0, ch, TILE):
            comm_ref[2, pl.ds(t, TILE), :] = (
                out_ref[pl.ds(t, TILE), :].astype(jnp.bfloat16)
            )

        send_slot = 2
        for h in range(NZ - 1):
            if h == NZ - 2:
                pl.semaphore_wait(credit_sem, 2)
            rdma_r = pltpu.make_async_remote_copy(
                src_ref=comm_ref.at[send_slot, pl.ds(0, hh), :],
                dst_ref=comm_ref.at[h, pl.ds(0, hh), :],
                send_sem=send_sems.at[h, 0],
                recv_sem=recv_sems.at[h, 0],
                device_id=(my_x, my_y, right),
                device_id_type=pl.DeviceIdType.MESH,
            )
            rdma_l = pltpu.make_async_remote_copy(
                src_ref=comm_ref.at[send_slot, pl.ds(hh, hh), :],
                dst_ref=comm_ref.at[h, pl.ds(hh, hh), :],
                send_sem=send_sems.at[h, 1],
                recv_sem=recv_sems.at[h, 1],
                device_id=(my_x, my_y, left),
                device_id_type=pl.DeviceIdType.MESH,
            )
            rdma_r.start()
            rdma_l.start()
            load_halves((my_z - h - 2) % NZ, (my_z + h + 2) % NZ)
            rdma_r.wait()
            rdma_l.wait()
            if h == 0:
                pl.semaphore_signal(
                    credit_sem, inc=1,
                    device_id=(my_x, my_y, left),
                    device_id_type=pl.DeviceIdType.MESH,
                )
                pl.semaphore_signal(
                    credit_sem, inc=1,
                    device_id=(my_x, my_y, right),
                    device_id_type=pl.DeviceIdType.MESH,
                )
            if h < NZ - 2:
                for t in range(0, ch, TILE):
                    comm_ref[h, pl.ds(t, TILE), :] = (
                        comm_ref[h, pl.ds(t, TILE), :].astype(jnp.float32)
                        + out_ref[pl.ds(t, TILE), :]
                    ).astype(jnp.bfloat16)
                send_slot = h
            else:
                g = g_ref[0, :][None, :]
                for t in range(0, ch, TILE):
                    acc = (
                        comm_ref[h, pl.ds(t, TILE), :].astype(jnp.float32)
                        + out_ref[pl.ds(t, TILE), :]
                    )
                    rms = jnp.sqrt(
                        jnp.mean(acc * acc, axis=-1, keepdims=True) + 1e-6
                    )
                    out_ref[pl.ds(t, TILE), :] = acc / rms * g

        @functools.partial(pl.run_scoped, sem=pltpu.SemaphoreType.REGULAR)
        def _(sem):
            for nbr in (left, right):
                pl.semaphore_signal(
                    sem, inc=1,
                    device_id=(my_x, my_y, nbr),
                    device_id_type=pl.DeviceIdType.MESH,
                )
            pl.semaphore_wait(sem, 2)

    return pl.pallas_call(
        body,
        out_shape=jax.ShapeDtypeStruct((ch, d), jnp.float32),
        in_specs=[
            pl.BlockSpec(memory_space=pl.ANY),
            pl.BlockSpec(memory_space=pltpu.VMEM),
        ],
        out_specs=pl.BlockSpec(memory_space=pltpu.VMEM),
        scratch_shapes=[
            pltpu.VMEM((NZ - 1, ch, d), jnp.bfloat16),
            pltpu.SemaphoreType.DMA((NZ - 1, 2)),
            pltpu.SemaphoreType.DMA((NZ - 1, 2)),
            pltpu.SemaphoreType.DMA((2,)),
            pltpu.SemaphoreType.REGULAR,
        ],
        compiler_params=pltpu.CompilerParams(
            collective_id=0,
            vmem_limit_bytes=63 * 1024 * 1024,
        ),
    )(partial, gamma.reshape(1, d))


# device time: 173704 ns/iter; 1.8160x vs baseline; 1.8160x over previous
import functools

import jax
import jax.numpy as jnp
from jax import lax
from jax.experimental import pallas as pl
from jax.experimental.pallas import tpu as pltpu

NZ = 4


def kernel(partial, gamma):
    _, m, d = partial.shape
    ch = m // NZ
    qh = ch // 4

    def body(x_ref, g_ref, out_ref, sbuf, rbuf, gath,
             z_send_sems, z_recv_sems, xy_send_sems, xy_recv_sems,
             copy_sems):
        my_x = lax.axis_index("x")
        my_y = lax.axis_index("y")
        my_z = lax.axis_index("z")
        p_me = my_x * 2 + my_y

        z_peers = [(my_x, my_y, (my_z + r) % NZ) for r in (1, 2, 3)]
        xy_offsets = [(1, 0), (0, 1), (1, 1)]
        xy_peers = [(my_x ^ dx, my_y ^ dy, my_z) for dx, dy in xy_offsets]

        copies = []
        for i, c in enumerate(
            [(my_z + r) % NZ for r in (1, 2, 3)] + [my_z]
        ):
            cp = pltpu.make_async_copy(
                x_ref.at[0, pl.ds(c * ch + p_me * qh, qh), :],
                out_ref.at[pl.ds(i * qh, qh), :],
                copy_sems.at[i],
            )
            cp.start()
            copies.append(cp)

        barrier = pltpu.get_barrier_semaphore()
        for nbr in z_peers + xy_peers:
            pl.semaphore_signal(
                barrier, inc=1, device_id=nbr,
                device_id_type=pl.DeviceIdType.MESH,
            )
        pl.semaphore_wait(barrier, 6)
        for cp in copies:
            cp.wait()

        for j in range(3):
            sbuf[j, :, :] = out_ref[pl.ds(j * qh, qh), :].astype(jnp.bfloat16)
        z_rdmas = []
        for j, peer in enumerate(z_peers):
            rdma = pltpu.make_async_remote_copy(
                src_ref=sbuf.at[j],
                dst_ref=rbuf.at[j],
                send_sem=z_send_sems.at[j],
                recv_sem=z_recv_sems.at[j],
                device_id=peer,
                device_id_type=pl.DeviceIdType.MESH,
            )
            rdma.start()
            z_rdmas.append(rdma)
        for rdma in z_rdmas:
            rdma.wait()

        qsum = out_ref[pl.ds(3 * qh, qh), :]
        for j in range(3):
            qsum = qsum + rbuf[j, :, :].astype(jnp.float32)
        gath[p_me, :, :] = qsum.astype(jnp.bfloat16)

        xy_rdmas = []
        for j, peer in enumerate(xy_peers):
            rdma = pltpu.make_async_remote_copy(
                src_ref=gath.at[p_me],
                dst_ref=gath.at[p_me],
                send_sem=xy_send_sems.at[j],
                recv_sem=xy_recv_sems.at[j],
                device_id=peer,
                device_id_type=pl.DeviceIdType.MESH,
            )
            rdma.start()
            xy_rdmas.append(rdma)
        for rdma in xy_rdmas:
            rdma.wait()

        g = g_ref[0, :][None, :]
        for q in range(4):
            acc = gath[q, :, :].astype(jnp.float32)
            rms = jnp.sqrt(jnp.mean(acc * acc, axis=-1, keepdims=True) + 1e-6)
            out_ref[pl.ds(q * qh, qh), :] = acc / rms * g

        @functools.partial(pl.run_scoped, sem=pltpu.SemaphoreType.REGULAR)
        def _(sem):
            for nbr in z_peers + xy_peers:
                pl.semaphore_signal(
                    sem, inc=1, device_id=nbr,
                    device_id_type=pl.DeviceIdType.MESH,
                )
            pl.semaphore_wait(sem, 6)

    return pl.pallas_call(
        body,
        out_shape=jax.ShapeDtypeStruct((ch, d), jnp.float32),
        in_specs=[
            pl.BlockSpec(memory_space=pl.ANY),
            pl.BlockSpec(memory_space=pltpu.VMEM),
        ],
        out_specs=pl.BlockSpec(memory_space=pltpu.VMEM),
        scratch_shapes=[
            pltpu.VMEM((3, qh, d), jnp.bfloat16),
            pltpu.VMEM((3, qh, d), jnp.bfloat16),
            pltpu.VMEM((4, qh, d), jnp.bfloat16),
            pltpu.SemaphoreType.DMA((3,)),
            pltpu.SemaphoreType.DMA((3,)),
            pltpu.SemaphoreType.DMA((3,)),
            pltpu.SemaphoreType.DMA((3,)),
            pltpu.SemaphoreType.DMA((4,)),
        ],
        compiler_params=pltpu.CompilerParams(
            collective_id=0,
            vmem_limit_bytes=63 * 1024 * 1024,
        ),
    )(partial, gamma.reshape(1, d))


# device time: 153014 ns/iter; 2.0615x vs baseline; 1.1352x over previous
import functools

import jax
import jax.numpy as jnp
from jax import lax
from jax.experimental import pallas as pl
from jax.experimental.pallas import tpu as pltpu

NZ = 4
S = 2


def kernel(partial, gamma):
    _, m, d = partial.shape
    ch = m // NZ
    qh = ch // 4

    def body(x_ref, g_ref, out_ref, sbuf, rbuf, gath,
             z_send_sems, z_recv_sems, xy_send_sems, xy_recv_sems,
             copy_sems):
        my_x = lax.axis_index("x")
        my_y = lax.axis_index("y")
        my_z = lax.axis_index("z")
        p_me = my_x * 2 + my_y

        z_peers = [(my_x, my_y, (my_z + r) % NZ) for r in (1, 2, 3)]
        xy_offsets = [(1, 0), (0, 1), (1, 1)]
        xy_peers = [(my_x ^ dx, my_y ^ dy, my_z) for dx, dy in xy_offsets]

        copies = []
        for i, c in enumerate(
            [(my_z + r) % NZ for r in (1, 2, 3)] + [my_z]
        ):
            cp = pltpu.make_async_copy(
                x_ref.at[0, pl.ds(c * ch + p_me * qh, qh), :],
                out_ref.at[pl.ds(i * qh, qh), :],
                copy_sems.at[i],
            )
            cp.start()
            copies.append(cp)

        barrier = pltpu.get_barrier_semaphore()
        for nbr in z_peers + xy_peers:
            pl.semaphore_signal(
                barrier, inc=1, device_id=nbr,
                device_id_type=pl.DeviceIdType.MESH,
            )
        pl.semaphore_wait(barrier, 6)
        for cp in copies:
            cp.wait()

        ph = qh // S

        def z_descriptor(j, i):
            return pltpu.make_async_remote_copy(
                src_ref=sbuf.at[j, pl.ds(i * ph, ph), :],
                dst_ref=rbuf.at[j, pl.ds(i * ph, ph), :],
                send_sem=z_send_sems.at[j, i],
                recv_sem=z_recv_sems.at[j, i],
                device_id=z_peers[j],
                device_id_type=pl.DeviceIdType.MESH,
            )

        def start_z_piece(i):
            for j in range(3):
                sbuf[j, pl.ds(i * ph, ph), :] = (
                    out_ref[pl.ds(j * qh + i * ph, ph), :].astype(jnp.bfloat16)
                )
            for j in range(3):
                z_descriptor(j, i).start()

        start_z_piece(0)
        xy_rdmas = []
        for i in range(S):
            for j in range(3):
                z_descriptor(j, i).wait()
            if i + 1 < S:
                start_z_piece(i + 1)
            rows = pl.ds(3 * qh + i * ph, ph)
            qsum = out_ref[rows, :]
            for j in range(3):
                qsum = qsum + rbuf[j, pl.ds(i * ph, ph), :].astype(jnp.float32)
            gath[p_me, pl.ds(i * ph, ph), :] = qsum.astype(jnp.bfloat16)

            for j, peer in enumerate(xy_peers):
                rdma = pltpu.make_async_remote_copy(
                    src_ref=gath.at[p_me, pl.ds(i * ph, ph), :],
                    dst_ref=gath.at[p_me, pl.ds(i * ph, ph), :],
                    send_sem=xy_send_sems.at[j, i],
                    recv_sem=xy_recv_sems.at[j, i],
                    device_id=peer,
                    device_id_type=pl.DeviceIdType.MESH,
                )
                rdma.start()
                xy_rdmas.append(rdma)
        for rdma in xy_rdmas:
            rdma.wait()

        g = g_ref[0, :][None, :]
        for q in range(4):
            acc = gath[q, :, :].astype(jnp.float32)
            rms = jnp.sqrt(jnp.mean(acc * acc, axis=-1, keepdims=True) + 1e-6)
            out_ref[pl.ds(q * qh, qh), :] = acc / rms * g

        @functools.partial(pl.run_scoped, sem=pltpu.SemaphoreType.REGULAR)
        def _(sem):
            for nbr in z_peers + xy_peers:
                pl.semaphore_signal(
                    sem, inc=1, device_id=nbr,
                    device_id_type=pl.DeviceIdType.MESH,
                )
            pl.semaphore_wait(sem, 6)

    return pl.pallas_call(
        body,
        out_shape=jax.ShapeDtypeStruct((ch, d), jnp.float32),
        in_specs=[
            pl.BlockSpec(memory_space=pl.ANY),
            pl.BlockSpec(memory_space=pltpu.VMEM),
        ],
        out_specs=pl.BlockSpec(memory_space=pltpu.VMEM),
        scratch_shapes=[
            pltpu.VMEM((3, qh, d), jnp.bfloat16),
            pltpu.VMEM((3, qh, d), jnp.bfloat16),
            pltpu.VMEM((4, qh, d), jnp.bfloat16),
            pltpu.SemaphoreType.DMA((3, S)),
            pltpu.SemaphoreType.DMA((3, S)),
            pltpu.SemaphoreType.DMA((3, S)),
            pltpu.SemaphoreType.DMA((3, S)),
            pltpu.SemaphoreType.DMA((4,)),
        ],
        compiler_params=pltpu.CompilerParams(
            collective_id=0,
            vmem_limit_bytes=63 * 1024 * 1024,
        ),
    )(partial, gamma.reshape(1, d))


# device time: 142908 ns/iter; 2.2073x vs baseline; 1.0707x over previous
import functools

import jax
import jax.numpy as jnp
from jax import lax
from jax.experimental import pallas as pl
from jax.experimental.pallas import tpu as pltpu

NZ = 4
S = 4


def kernel(partial, gamma):
    _, m, d = partial.shape
    ch = m // NZ
    qh = ch // 4

    def body(x_ref, g_ref, out_ref, sbuf, rbuf, gath,
             z_send_sems, z_recv_sems, xy_send_sems, xy_recv_sems,
             copy_sems):
        my_x = lax.axis_index("x")
        my_y = lax.axis_index("y")
        my_z = lax.axis_index("z")
        p_me = my_x * 2 + my_y

        z_peers = [(my_x, my_y, (my_z + r) % NZ) for r in (1, 2, 3)]
        xy_offsets = [(1, 0), (0, 1), (1, 1)]
        xy_peers = [(my_x ^ dx, my_y ^ dy, my_z) for dx, dy in xy_offsets]

        copies = []
        for i, c in enumerate(
            [(my_z + r) % NZ for r in (1, 2, 3)] + [my_z]
        ):
            cp = pltpu.make_async_copy(
                x_ref.at[0, pl.ds(c * ch + p_me * qh, qh), :],
                out_ref.at[pl.ds(i * qh, qh), :],
                copy_sems.at[i],
            )
            cp.start()
            copies.append(cp)

        barrier = pltpu.get_barrier_semaphore()
        for nbr in z_peers + xy_peers:
            pl.semaphore_signal(
                barrier, inc=1, device_id=nbr,
                device_id_type=pl.DeviceIdType.MESH,
            )
        pl.semaphore_wait(barrier, 6)
        for cp in copies:
            cp.wait()

        ph = qh // S

        def z_descriptor(j, i):
            return pltpu.make_async_remote_copy(
                src_ref=sbuf.at[j, pl.ds(i * ph, ph), :],
                dst_ref=rbuf.at[j, pl.ds(i * ph, ph), :],
                send_sem=z_send_sems.at[j, i],
                recv_sem=z_recv_sems.at[j, i],
                device_id=z_peers[j],
                device_id_type=pl.DeviceIdType.MESH,
            )

        def start_z_piece(i):
            for j in range(3):
                sbuf[j, pl.ds(i * ph, ph), :] = (
                    out_ref[pl.ds(j * qh + i * ph, ph), :].astype(jnp.bfloat16)
                )
            for j in range(3):
                z_descriptor(j, i).start()

        start_z_piece(0)
        xy_rdmas = []
        for i in range(S):
            for j in range(3):
                z_descriptor(j, i).wait()
            if i + 1 < S:
                start_z_piece(i + 1)
            rows = pl.ds(3 * qh + i * ph, ph)
            qsum = out_ref[rows, :]
            for j in range(3):
                qsum = qsum + rbuf[j, pl.ds(i * ph, ph), :].astype(jnp.float32)
            gath[p_me, pl.ds(i * ph, ph), :] = qsum.astype(jnp.bfloat16)

            for j, peer in enumerate(xy_peers):
                rdma = pltpu.make_async_remote_copy(
                    src_ref=gath.at[p_me, pl.ds(i * ph, ph), :],
                    dst_ref=gath.at[p_me, pl.ds(i * ph, ph), :],
                    send_sem=xy_send_sems.at[j, i],
                    recv_sem=xy_recv_sems.at[j, i],
                    device_id=peer,
                    device_id_type=pl.DeviceIdType.MESH,
                )
                rdma.start()
                xy_rdmas.append(rdma)
        for rdma in xy_rdmas:
            rdma.wait()

        g = g_ref[0, :][None, :]
        for q in range(4):
            acc = gath[q, :, :].astype(jnp.float32)
            rms = jnp.sqrt(jnp.mean(acc * acc, axis=-1, keepdims=True) + 1e-6)
            out_ref[pl.ds(q * qh, qh), :] = acc / rms * g

        @functools.partial(pl.run_scoped, sem=pltpu.SemaphoreType.REGULAR)
        def _(sem):
            for nbr in z_peers + xy_peers:
                pl.semaphore_signal(
                    sem, inc=1, device_id=nbr,
                    device_id_type=pl.DeviceIdType.MESH,
                )
            pl.semaphore_wait(sem, 6)

    return pl.pallas_call(
        body,
        out_shape=jax.ShapeDtypeStruct((ch, d), jnp.float32),
        in_specs=[
            pl.BlockSpec(memory_space=pl.ANY),
            pl.BlockSpec(memory_space=pltpu.VMEM),
        ],
        out_specs=pl.BlockSpec(memory_space=pltpu.VMEM),
        scratch_shapes=[
            pltpu.VMEM((3, qh, d), jnp.bfloat16),
            pltpu.VMEM((3, qh, d), jnp.bfloat16),
            pltpu.VMEM((4, qh, d), jnp.bfloat16),
            pltpu.SemaphoreType.DMA((3, S)),
            pltpu.SemaphoreType.DMA((3, S)),
            pltpu.SemaphoreType.DMA((3, S)),
            pltpu.SemaphoreType.DMA((3, S)),
            pltpu.SemaphoreType.DMA((4,)),
        ],
        compiler_params=pltpu.CompilerParams(
            collective_id=0,
            vmem_limit_bytes=63 * 1024 * 1024,
        ),
    )(partial, gamma.reshape(1, d))


# device time: 139233 ns/iter; 2.2656x vs baseline; 1.0264x over previous
import functools

import jax
import jax.numpy as jnp
from jax import lax
from jax.experimental import pallas as pl
from jax.experimental.pallas import tpu as pltpu

NZ = 4
S = 8


def kernel(partial, gamma):
    _, m, d = partial.shape
    ch = m // NZ
    qh = ch // 4

    def body(x_ref, g_ref, out_ref, sbuf, rbuf, gath,
             z_send_sems, z_recv_sems, xy_send_sems, xy_recv_sems,
             copy_sems):
        my_x = lax.axis_index("x")
        my_y = lax.axis_index("y")
        my_z = lax.axis_index("z")
        p_me = my_x * 2 + my_y

        z_peers = [(my_x, my_y, (my_z + r) % NZ) for r in (1, 2, 3)]
        xy_offsets = [(1, 0), (0, 1), (1, 1)]
        xy_peers = [(my_x ^ dx, my_y ^ dy, my_z) for dx, dy in xy_offsets]

        copies = []
        for i, c in enumerate(
            [(my_z + r) % NZ for r in (1, 2, 3)] + [my_z]
        ):
            cp = pltpu.make_async_copy(
                x_ref.at[0, pl.ds(c * ch + p_me * qh, qh), :],
                out_ref.at[pl.ds(i * qh, qh), :],
                copy_sems.at[i],
            )
            cp.start()
            copies.append(cp)

        barrier = pltpu.get_barrier_semaphore()
        for nbr in z_peers + xy_peers:
            pl.semaphore_signal(
                barrier, inc=1, device_id=nbr,
                device_id_type=pl.DeviceIdType.MESH,
            )
        pl.semaphore_wait(barrier, 6)
        for cp in copies:
            cp.wait()

        ph = qh // S

        def z_descriptor(j, i):
            return pltpu.make_async_remote_copy(
                src_ref=sbuf.at[j, pl.ds(i * ph, ph), :],
                dst_ref=rbuf.at[j, pl.ds(i * ph, ph), :],
                send_sem=z_send_sems.at[j, i],
                recv_sem=z_recv_sems.at[j, i],
                device_id=z_peers[j],
                device_id_type=pl.DeviceIdType.MESH,
            )

        def start_z_piece(i):
            for j in range(3):
                sbuf[j, pl.ds(i * ph, ph), :] = (
                    out_ref[pl.ds(j * qh + i * ph, ph), :].astype(jnp.bfloat16)
                )
            for j in range(3):
                z_descriptor(j, i).start()

        start_z_piece(0)
        xy_rdmas = []
        for i in range(S):
            for j in range(3):
                z_descriptor(j, i).wait()
            if i + 1 < S:
                start_z_piece(i + 1)
            rows = pl.ds(3 * qh + i * ph, ph)
            qsum = out_ref[rows, :]
            for j in range(3):
                qsum = qsum + rbuf[j, pl.ds(i * ph, ph), :].astype(jnp.float32)
            gath[p_me, pl.ds(i * ph, ph), :] = qsum.astype(jnp.bfloat16)

            for j, peer in enumerate(xy_peers):
                rdma = pltpu.make_async_remote_copy(
                    src_ref=gath.at[p_me, pl.ds(i * ph, ph), :],
                    dst_ref=gath.at[p_me, pl.ds(i * ph, ph), :],
                    send_sem=xy_send_sems.at[j, i],
                    recv_sem=xy_recv_sems.at[j, i],
                    device_id=peer,
                    device_id_type=pl.DeviceIdType.MESH,
                )
                rdma.start()
                xy_rdmas.append(rdma)

        g = g_ref[0, :][None, :]
        for i in range(S):
            for rdma in xy_rdmas[3 * i : 3 * i + 3]:
                rdma.wait()
            for q in range(4):
                rows = pl.ds(q * qh + i * ph, ph)
                acc = gath[q, pl.ds(i * ph, ph), :].astype(jnp.float32)
                rms = jnp.sqrt(
                    jnp.mean(acc * acc, axis=-1, keepdims=True) + 1e-6
                )
                out_ref[rows, :] = acc / rms * g

        @functools.partial(pl.run_scoped, sem=pltpu.SemaphoreType.REGULAR)
        def _(sem):
            for nbr in z_peers + xy_peers:
                pl.semaphore_signal(
                    sem, inc=1, device_id=nbr,
                    device_id_type=pl.DeviceIdType.MESH,
                )
            pl.semaphore_wait(sem, 6)

    return pl.pallas_call(
        body,
        out_shape=jax.ShapeDtypeStruct((ch, d), jnp.float32),
        in_specs=[
            pl.BlockSpec(memory_space=pl.ANY),
            pl.BlockSpec(memory_space=pltpu.VMEM),
        ],
        out_specs=pl.BlockSpec(memory_space=pltpu.VMEM),
        scratch_shapes=[
            pltpu.VMEM((3, qh, d), jnp.bfloat16),
            pltpu.VMEM((3, qh, d), jnp.bfloat16),
            pltpu.VMEM((4, qh, d), jnp.bfloat16),
            pltpu.SemaphoreType.DMA((3, S)),
            pltpu.SemaphoreType.DMA((3, S)),
            pltpu.SemaphoreType.DMA((3, S)),
            pltpu.SemaphoreType.DMA((3, S)),
            pltpu.SemaphoreType.DMA((4,)),
        ],
        compiler_params=pltpu.CompilerParams(
            collective_id=0,
            vmem_limit_bytes=63 * 1024 * 1024,
        ),
    )(partial, gamma.reshape(1, d))
